# baseline (speedup 1.0000x reference)
import jax
import jax.numpy as jnp
from jax import lax
from jax.experimental import pallas as pl
from jax.experimental.pallas import tpu as pltpu
from jax.experimental.pallas import tpu_sc as plsc

N_CORES = 2
N_SUBCORES = 16
N_WORKERS = N_CORES * N_SUBCORES
LANES = 16


def _gather_body(main_x, main_t, tail_x, tail_t, idx_hbm, idxc_hbm,
                 out_x_hbm, out_t_hbm,
                 idx_v, idxc_v, rows_x, rows_t, tailx_v, tailt_v,
                 sem_x, sem_t, sem_i, sem_tl):
    m = main_x.shape[0]
    b_per_w = idx_v.shape[0]
    wid = lax.axis_index("s") * N_CORES + lax.axis_index("c")
    base = wid * b_per_w
    ci = pltpu.async_copy(idx_hbm.at[pl.ds(base, b_per_w)], idx_v, sem_i)
    ctl_x = pltpu.async_copy(tail_x, tailx_v, sem_tl)
    ctl_t = pltpu.async_copy(tail_t, tailt_v, sem_tl)
    pltpu.sync_copy(idxc_hbm.at[pl.ds(base, b_per_w)], idxc_v)
    cx = pltpu.async_copy(main_x.at[idxc_v], rows_x, sem_x)
    ct = pltpu.async_copy(main_t.at[idxc_v], rows_t, sem_t)
    ci.wait()
    ctl_x.wait()
    ctl_t.wait()
    cx.wait()
    ct.wait()

    def patch(i, _):
        o = i * LANES
        vi = idx_v[pl.ds(o, LANES)]
        cond = vi >= m
        ti = jnp.maximum(vi - m, 0)
        vx = plsc.load_gather(tailx_v, [ti])
        vt = plsc.load_gather(tailt_v, [ti])
        rows_x[pl.ds(o, LANES)] = jnp.where(cond, vx, rows_x[pl.ds(o, LANES)])
        rows_t[pl.ds(o, LANES)] = jnp.where(cond, vt, rows_t[pl.ds(o, LANES)])
        return _

    lax.fori_loop(0, b_per_w // LANES, patch, None)

    pltpu.sync_copy(rows_x, out_x_hbm.at[pl.ds(base, b_per_w)])
    pltpu.sync_copy(rows_t, out_t_hbm.at[pl.ds(base, b_per_w)])


def kernel(x, t, idx):
    n_rows = x.shape[0]
    n = idx.shape[0]
    m = (n_rows // 1024) * 1024
    n_tail = n_rows - m
    b_per_w = -(-n // (LANES * N_WORKERS)) * LANES
    n_pad = b_per_w * N_WORKERS
    idx32 = idx.astype(jnp.int32)
    if n_pad != n:
        idx32 = jnp.concatenate(
            [idx32, jnp.zeros((n_pad - n,), dtype=jnp.int32)])
    idxc32 = jnp.minimum(idx32, m - 1)

    main_x = x[:m].reshape(-1)
    main_t = t[:m].reshape(-1)
    tail_x = x[m:].reshape(-1)
    tail_t = t[m:].reshape(-1)

    mesh = plsc.VectorSubcoreMesh(
        core_axis_name="c", subcore_axis_name="s",
        num_cores=N_CORES, num_subcores=N_SUBCORES)
    out_x, out_t = pl.kernel(
        _gather_body,
        out_type=(
            jax.ShapeDtypeStruct((n_pad,), jnp.float32),
            jax.ShapeDtypeStruct((n_pad,), jnp.float32),
        ),
        mesh=mesh,
        scratch_types=[
            pltpu.VMEM((b_per_w,), jnp.int32),
            pltpu.VMEM((b_per_w,), jnp.int32),
            pltpu.VMEM((b_per_w,), jnp.float32),
            pltpu.VMEM((b_per_w,), jnp.float32),
            pltpu.VMEM((n_tail,), jnp.float32),
            pltpu.VMEM((n_tail,), jnp.float32),
            pltpu.SemaphoreType.DMA,
            pltpu.SemaphoreType.DMA,
            pltpu.SemaphoreType.DMA,
            pltpu.SemaphoreType.DMA,
        ],
        name="holdout_sampler_gather",
        compiler_params=pltpu.CompilerParams(needs_layout_passes=False),
    )(main_x, main_t, tail_x, tail_t, idx32, idxc32)

    return (out_x[:n].reshape(n, 1), out_t[:n].reshape(n, 1))
